# Initial kernel scaffold; baseline (speedup 1.0000x reference)
import functools

import jax
import jax.numpy as jnp
from jax import lax
from jax.experimental import pallas as pl
from jax.experimental.pallas import tpu as pltpu
from jax.experimental.pallas import tpu_sc as plsc

BATCH = 16384
TITLE_DIM = 200
NUM_DIM = 36
DOMAIN_DIM = 16
USER_DIM = 24
HIDDEN = 128

NC = 2
NS = 16
NW = NC * NS
B_PER_W = BATCH // NW


def _make_gather():
    mesh = plsc.VectorSubcoreMesh(core_axis_name="c", subcore_axis_name="s")

    @functools.partial(
        pl.kernel,
        mesh=mesh,
        out_type=[
            jax.ShapeDtypeStruct((BATCH, DOMAIN_DIM), jnp.float32),
            jax.ShapeDtypeStruct((BATCH, USER_DIM), jnp.float32),
        ],
        scratch_types=[
            pltpu.VMEM((B_PER_W,), jnp.int32),
            pltpu.VMEM((B_PER_W,), jnp.int32),
            pltpu.VMEM((B_PER_W, DOMAIN_DIM), jnp.float32),
            pltpu.VMEM((B_PER_W, USER_DIM), jnp.float32),
            pltpu.SemaphoreType.DMA,
            pltpu.SemaphoreType.DMA,
        ],
    )
    def gather_kernel(dtab_hbm, utab_hbm, dids_hbm, uids_hbm,
                      dom_out, usr_out,
                      didx_v, uidx_v, drows_v, urows_v, dsem, usem):
        wid = lax.axis_index("s") * NC + lax.axis_index("c")
        base = wid * B_PER_W
        pltpu.sync_copy(dids_hbm.at[pl.ds(base, B_PER_W)], didx_v)
        pltpu.sync_copy(uids_hbm.at[pl.ds(base, B_PER_W)], uidx_v)
        dcp = pltpu.async_copy(dtab_hbm.at[didx_v], drows_v, dsem)
        ucp = pltpu.async_copy(utab_hbm.at[uidx_v], urows_v, usem)
        dcp.wait()
        ucp.wait()
        pltpu.sync_copy(drows_v, dom_out.at[pl.ds(base, B_PER_W)])
        pltpu.sync_copy(urows_v, usr_out.at[pl.ds(base, B_PER_W)])

    return gather_kernel


_gather = _make_gather()


def _mlp_body(title_ref, num_ref, dom_ref, usr_ref,
              w1t_ref, w1n_ref, w1d_ref, w1u_ref, b1_ref,
              w2_ref, b2_ref, w3t_ref, b3_ref, out_ref):
    acc = jnp.dot(title_ref[...], w1t_ref[...], preferred_element_type=jnp.float32)
    acc += jnp.dot(num_ref[...], w1n_ref[...], preferred_element_type=jnp.float32)
    acc += jnp.dot(dom_ref[...], w1d_ref[...], preferred_element_type=jnp.float32)
    acc += jnp.dot(usr_ref[...], w1u_ref[...], preferred_element_type=jnp.float32)
    h1 = jnp.maximum(acc + b1_ref[...][None, :], 0.0)
    h2 = jnp.maximum(
        jnp.dot(h1, w2_ref[...], preferred_element_type=jnp.float32)
        + b2_ref[...][None, :], 0.0)
    out_ref[...] = jnp.sum(h2 * w3t_ref[...], axis=1) + b3_ref[...]


def _mlp(title_emb, numerical_features, dom_emb, usr_emb,
         w1t, w1n, w1d, w1u, b1, w2, b2, w3t, b3, block_m=2048):
    grid = (BATCH // block_m,)

    def full(a):
        return pl.BlockSpec(a.shape, lambda i: (0,) * a.ndim)

    return pl.pallas_call(
        _mlp_body,
        grid=grid,
        in_specs=[
            pl.BlockSpec((block_m, TITLE_DIM), lambda i: (i, 0)),
            pl.BlockSpec((block_m, NUM_DIM), lambda i: (i, 0)),
            pl.BlockSpec((block_m, DOMAIN_DIM), lambda i: (i, 0)),
            pl.BlockSpec((block_m, USER_DIM), lambda i: (i, 0)),
            full(w1t), full(w1n), full(w1d), full(w1u), full(b1),
            full(w2), full(b2), full(w3t), full(b3),
        ],
        out_specs=pl.BlockSpec((block_m,), lambda i: (i,)),
        out_shape=jax.ShapeDtypeStruct((BATCH,), jnp.float32),
        compiler_params=pltpu.CompilerParams(
            dimension_semantics=("parallel",)),
    )(title_emb, numerical_features, dom_emb, usr_emb,
      w1t, w1n, w1d, w1u, b1, w2, b2, w3t, b3)


def kernel(title_emb, numerical_features, domain_ids, user_ids,
           domain_table, user_table, W1, b1, W2, b2, W3, b3):
    dom_emb, usr_emb = _gather(domain_table, user_table, domain_ids, user_ids)
    w1t = W1[:TITLE_DIM] * 0.5
    w1n = W1[TITLE_DIM:TITLE_DIM + NUM_DIM]
    w1d = W1[TITLE_DIM + NUM_DIM:TITLE_DIM + NUM_DIM + DOMAIN_DIM]
    w1u = W1[TITLE_DIM + NUM_DIM + DOMAIN_DIM:]
    w3t = W3.reshape(1, -1)
    return _mlp(title_emb, numerical_features, dom_emb, usr_emb,
                w1t, w1n, w1d, w1u, b1, W2, b2, w3t, b3)


# speedup vs baseline: 3.0719x; 3.0719x over previous
import functools

import jax
import jax.numpy as jnp
from jax import lax
from jax.experimental import pallas as pl
from jax.experimental.pallas import tpu as pltpu
from jax.experimental.pallas import tpu_sc as plsc

BATCH = 16384
TITLE_DIM = 200
NUM_DIM = 36
DOMAIN_DIM = 16
USER_DIM = 24
HIDDEN = 128

NC = 2
NS = 16
NW = NC * NS
B_PER_W = BATCH // NW


def _make_gather():
    mesh = plsc.VectorSubcoreMesh(core_axis_name="c", subcore_axis_name="s")

    @functools.partial(
        pl.kernel,
        mesh=mesh,
        out_type=[
            jax.ShapeDtypeStruct((BATCH, DOMAIN_DIM), jnp.float32),
            jax.ShapeDtypeStruct((BATCH, USER_DIM), jnp.float32),
        ],
        scratch_types=[
            pltpu.VMEM((B_PER_W,), jnp.int32),
            pltpu.VMEM((B_PER_W,), jnp.int32),
            pltpu.VMEM((B_PER_W, DOMAIN_DIM), jnp.float32),
            pltpu.VMEM((B_PER_W, USER_DIM), jnp.float32),
            pltpu.SemaphoreType.DMA,
            pltpu.SemaphoreType.DMA,
        ],
    )
    def gather_kernel(dtab_hbm, utab_hbm, dids_hbm, uids_hbm,
                      dom_out, usr_out,
                      didx_v, uidx_v, drows_v, urows_v, dsem, usem):
        wid = lax.axis_index("s") * NC + lax.axis_index("c")
        base = wid * B_PER_W
        pltpu.sync_copy(dids_hbm.at[pl.ds(base, B_PER_W)], didx_v)
        pltpu.sync_copy(uids_hbm.at[pl.ds(base, B_PER_W)], uidx_v)
        dcp = pltpu.async_copy(dtab_hbm.at[didx_v], drows_v, dsem)
        ucp = pltpu.async_copy(utab_hbm.at[uidx_v], urows_v, usem)
        dcp.wait()
        ucp.wait()
        pltpu.sync_copy(drows_v, dom_out.at[pl.ds(base, B_PER_W)])
        pltpu.sync_copy(urows_v, usr_out.at[pl.ds(base, B_PER_W)])

    return gather_kernel


_gather = _make_gather()


def _mlp_body(title_ref, num_ref, dom_ref, usr_ref,
              w1t_ref, w1n_ref, w1d_ref, w1u_ref, b1_ref,
              w2_ref, b2_ref, w3t_ref, b3_ref, out_ref):
    acc = jnp.dot(title_ref[...], w1t_ref[...], preferred_element_type=jnp.float32)
    acc += jnp.dot(num_ref[...], w1n_ref[...], preferred_element_type=jnp.float32)
    acc += jnp.dot(dom_ref[...], w1d_ref[...], preferred_element_type=jnp.float32)
    acc += jnp.dot(usr_ref[...], w1u_ref[...], preferred_element_type=jnp.float32)
    h1 = jnp.maximum(acc + b1_ref[...][None, :], 0.0)
    h2 = jnp.maximum(
        jnp.dot(h1, w2_ref[...], preferred_element_type=jnp.float32)
        + b2_ref[...][None, :], 0.0)
    out_ref[...] = jnp.sum(h2 * w3t_ref[...], axis=1) + b3_ref[...]


def _mlp(title_emb, numerical_features, dom_emb, usr_emb,
         w1t, w1n, w1d, w1u, b1, w2, b2, w3t, b3, block_m=2048):
    grid = (BATCH // block_m,)

    def full(a):
        return pl.BlockSpec(a.shape, lambda i: (0,) * a.ndim)

    return pl.pallas_call(
        _mlp_body,
        grid=grid,
        in_specs=[
            pl.BlockSpec((block_m, TITLE_DIM), lambda i: (i, 0)),
            pl.BlockSpec((block_m, NUM_DIM), lambda i: (i, 0)),
            pl.BlockSpec((block_m, DOMAIN_DIM), lambda i: (i, 0)),
            pl.BlockSpec((block_m, USER_DIM), lambda i: (i, 0)),
            full(w1t), full(w1n), full(w1d), full(w1u), full(b1),
            full(w2), full(b2), full(w3t), full(b3),
        ],
        out_specs=pl.BlockSpec((block_m,), lambda i: (i,)),
        out_shape=jax.ShapeDtypeStruct((BATCH,), jnp.float32),
        compiler_params=pltpu.CompilerParams(
            dimension_semantics=("parallel",)),
    )(title_emb, numerical_features, dom_emb, usr_emb,
      w1t, w1n, w1d, w1u, b1, w2, b2, w3t, b3)


def kernel(title_emb, numerical_features, domain_ids, user_ids,
           domain_table, user_table, W1, b1, W2, b2, W3, b3):
    dom_emb = jnp.take(domain_table, domain_ids, axis=0)
    usr_emb = jnp.take(user_table, user_ids, axis=0)
    w1t = W1[:TITLE_DIM] * 0.5
    w1n = W1[TITLE_DIM:TITLE_DIM + NUM_DIM]
    w1d = W1[TITLE_DIM + NUM_DIM:TITLE_DIM + NUM_DIM + DOMAIN_DIM]
    w1u = W1[TITLE_DIM + NUM_DIM + DOMAIN_DIM:]
    w3t = W3.reshape(1, -1)
    return _mlp(title_emb, numerical_features, dom_emb, usr_emb,
                w1t, w1n, w1d, w1u, b1, W2, b2, w3t, b3)


# speedup vs baseline: 3.2786x; 1.0673x over previous
import functools

import jax
import jax.numpy as jnp
from jax import lax
from jax.experimental import pallas as pl
from jax.experimental.pallas import tpu as pltpu
from jax.experimental.pallas import tpu_sc as plsc

BATCH = 16384
TITLE_DIM = 200
NUM_DIM = 36
DOMAIN_DIM = 16
USER_DIM = 24
HIDDEN = 128

NC = 2
NS = 16
NW = NC * NS
B_PER_W = BATCH // NW


def _make_gather():
    mesh = plsc.VectorSubcoreMesh(core_axis_name="c", subcore_axis_name="s")

    @functools.partial(
        pl.kernel,
        mesh=mesh,
        out_type=[
            jax.ShapeDtypeStruct((BATCH, DOMAIN_DIM), jnp.float32),
            jax.ShapeDtypeStruct((BATCH, USER_DIM), jnp.float32),
        ],
        scratch_types=[
            pltpu.VMEM((B_PER_W,), jnp.int32),
            pltpu.VMEM((B_PER_W,), jnp.int32),
            pltpu.VMEM((B_PER_W, DOMAIN_DIM), jnp.float32),
            pltpu.VMEM((B_PER_W, USER_DIM), jnp.float32),
            pltpu.SemaphoreType.DMA,
            pltpu.SemaphoreType.DMA,
        ],
    )
    def gather_kernel(dtab_hbm, utab_hbm, dids_hbm, uids_hbm,
                      dom_out, usr_out,
                      didx_v, uidx_v, drows_v, urows_v, dsem, usem):
        wid = lax.axis_index("s") * NC + lax.axis_index("c")
        base = wid * B_PER_W
        pltpu.sync_copy(dids_hbm.at[pl.ds(base, B_PER_W)], didx_v)
        pltpu.sync_copy(uids_hbm.at[pl.ds(base, B_PER_W)], uidx_v)
        dcp = pltpu.async_copy(dtab_hbm.at[didx_v], drows_v, dsem)
        ucp = pltpu.async_copy(utab_hbm.at[uidx_v], urows_v, usem)
        dcp.wait()
        ucp.wait()
        pltpu.sync_copy(drows_v, dom_out.at[pl.ds(base, B_PER_W)])
        pltpu.sync_copy(urows_v, usr_out.at[pl.ds(base, B_PER_W)])

    return gather_kernel


_gather = _make_gather()


def _mlp_body(title_ref, num_ref, dom_ref, usr_ref,
              w1t_ref, w1n_ref, w1d_ref, w1u_ref, b1_ref,
              w2_ref, b2_ref, w3t_ref, b3_ref, out_ref):
    bf = jnp.bfloat16
    acc = jnp.dot(title_ref[...].astype(bf), w1t_ref[...].astype(bf),
                  preferred_element_type=jnp.float32)
    acc += jnp.dot(num_ref[...].astype(bf), w1n_ref[...].astype(bf),
                   preferred_element_type=jnp.float32)
    acc += jnp.dot(dom_ref[...].astype(bf), w1d_ref[...].astype(bf),
                   preferred_element_type=jnp.float32)
    acc += jnp.dot(usr_ref[...].astype(bf), w1u_ref[...].astype(bf),
                   preferred_element_type=jnp.float32)
    h1 = jnp.maximum(acc + b1_ref[...][None, :], 0.0)
    h2 = jnp.maximum(
        jnp.dot(h1.astype(bf), w2_ref[...].astype(bf),
                preferred_element_type=jnp.float32)
        + b2_ref[...][None, :], 0.0)
    out_ref[...] = jnp.sum(h2 * w3t_ref[...], axis=1) + b3_ref[...]


def _mlp(title_emb, numerical_features, dom_emb, usr_emb,
         w1t, w1n, w1d, w1u, b1, w2, b2, w3t, b3, block_m=2048):
    grid = (BATCH // block_m,)

    def full(a):
        return pl.BlockSpec(a.shape, lambda i: (0,) * a.ndim)

    return pl.pallas_call(
        _mlp_body,
        grid=grid,
        in_specs=[
            pl.BlockSpec((block_m, TITLE_DIM), lambda i: (i, 0)),
            pl.BlockSpec((block_m, NUM_DIM), lambda i: (i, 0)),
            pl.BlockSpec((block_m, DOMAIN_DIM), lambda i: (i, 0)),
            pl.BlockSpec((block_m, USER_DIM), lambda i: (i, 0)),
            full(w1t), full(w1n), full(w1d), full(w1u), full(b1),
            full(w2), full(b2), full(w3t), full(b3),
        ],
        out_specs=pl.BlockSpec((block_m,), lambda i: (i,)),
        out_shape=jax.ShapeDtypeStruct((BATCH,), jnp.float32),
        compiler_params=pltpu.CompilerParams(
            dimension_semantics=("parallel",)),
    )(title_emb, numerical_features, dom_emb, usr_emb,
      w1t, w1n, w1d, w1u, b1, w2, b2, w3t, b3)


def kernel(title_emb, numerical_features, domain_ids, user_ids,
           domain_table, user_table, W1, b1, W2, b2, W3, b3):
    dom_emb = jnp.take(domain_table, domain_ids, axis=0)
    usr_emb = jnp.take(user_table, user_ids, axis=0)
    w1t = W1[:TITLE_DIM] * 0.5
    w1n = W1[TITLE_DIM:TITLE_DIM + NUM_DIM]
    w1d = W1[TITLE_DIM + NUM_DIM:TITLE_DIM + NUM_DIM + DOMAIN_DIM]
    w1u = W1[TITLE_DIM + NUM_DIM + DOMAIN_DIM:]
    w3t = W3.reshape(1, -1)
    return _mlp(title_emb, numerical_features, dom_emb, usr_emb,
                w1t, w1n, w1d, w1u, b1, W2, b2, w3t, b3)
